# baseline (speedup 1.0000x reference)
import functools

import jax
import jax.numpy as jnp
from jax import lax
from jax.experimental import pallas as pl
from jax.experimental.pallas import tpu as pltpu
from jax.experimental.pallas import tpu_sc as plsc

M = 1000000
N = 1000000
K = 16
B = 16384

_NC = 2
_NS = 16
_NW = _NC * _NS
_BPW = B // _NW


def _funk_body(uid, iid, uemb, ubias, iemb, ibias, bias16, out,
               uidx_v, iidx_v, ue_v, ie_v, ub_v, ib_v, bias_v, out_v, sem):
  wid = lax.axis_index("s") * _NC + lax.axis_index("c")
  base = wid * _BPW

  pltpu.sync_copy(uid.at[pl.ds(base, _BPW)], uidx_v)
  pltpu.sync_copy(iid.at[pl.ds(base, _BPW)], iidx_v)
  pltpu.sync_copy(bias16, bias_v)

  cps = [
      pltpu.async_copy(uemb.at[uidx_v], ue_v, sem),
      pltpu.async_copy(iemb.at[iidx_v], ie_v, sem),
      pltpu.async_copy(ubias.at[uidx_v], ub_v, sem),
      pltpu.async_copy(ibias.at[iidx_v], ib_v, sem),
  ]
  for cp in cps:
    cp.wait()

  bvec = bias_v[...]

  def chunk(c, carry):
    ub16 = ub_v[pl.ds(c * 16, 16)]
    ib16 = ib_v[pl.ds(c * 16, 16)]
    for lane in range(16):
      b = c * 16 + lane
      out_v[b, :] = (ue_v[b, :] + ub16[lane]) * (ie_v[b, :] + ib16[lane]) + bvec
    return carry

  lax.fori_loop(0, _BPW // 16, chunk, 0)

  pltpu.sync_copy(out_v, out.at[pl.ds(base, _BPW)])


@functools.partial(
    pl.kernel,
    out_type=jax.ShapeDtypeStruct((B, K), jnp.float32),
    mesh=plsc.VectorSubcoreMesh(core_axis_name="c", subcore_axis_name="s"),
    scratch_types=[
        pltpu.VMEM((_BPW,), jnp.int32),
        pltpu.VMEM((_BPW,), jnp.int32),
        pltpu.VMEM((_BPW, K), jnp.float32),
        pltpu.VMEM((_BPW, K), jnp.float32),
        pltpu.VMEM((_BPW,), jnp.float32),
        pltpu.VMEM((_BPW,), jnp.float32),
        pltpu.VMEM((16,), jnp.float32),
        pltpu.VMEM((_BPW, K), jnp.float32),
        pltpu.SemaphoreType.DMA,
    ],
    compiler_params=pltpu.CompilerParams(use_tc_tiling_on_sc=False),
)
def _funk(*args):
  _funk_body(*args)


def kernel(user_id, item_id, user_emb, user_bias, item_emb, item_bias, bias):
  bias16 = jnp.broadcast_to(bias.astype(jnp.float32), (16,))
  return _funk(user_id.astype(jnp.int32), item_id.astype(jnp.int32),
               user_emb, user_bias, item_emb, item_bias, bias16)


# speedup vs baseline: 1.4836x; 1.4836x over previous
import functools

import jax
import jax.numpy as jnp
from jax import lax
from jax.experimental import pallas as pl
from jax.experimental.pallas import tpu as pltpu
from jax.experimental.pallas import tpu_sc as plsc

M = 1000000
N = 1000000
K = 16
B = 16384

_NC = 2
_NS = 16
_NW = _NC * _NS
_BPW = B // _NW
_CH = 256


def _funk_body(uid, iid, uemb, ubias, iemb, ibias, bias128, out,
               uidx_v, iidx_v, ue_t, ie_t, ub_v, ib_v, bias_v, out_t, sem):
  wid = lax.axis_index("s") * _NC + lax.axis_index("c")
  base = wid * _BPW

  pltpu.sync_copy(uid.at[pl.ds(base, _BPW)], uidx_v)
  pltpu.sync_copy(iid.at[pl.ds(base, _BPW)], iidx_v)
  pltpu.sync_copy(bias128, bias_v)

  bcp1 = pltpu.async_copy(ubias.at[uidx_v], ub_v, sem)
  bcp2 = pltpu.async_copy(ibias.at[iidx_v], ib_v, sem)
  bcp1.wait()
  bcp2.wait()

  bvec = bias_v[pl.ds(0, 16)]

  for half in range(2):
    off = half * _CH

    def issue(g, carry):
      uidx16 = uidx_v[pl.ds(off + g * 16, 16)]
      iidx16 = iidx_v[pl.ds(off + g * 16, 16)]
      for lane in range(16):
        j = g * 16 + lane
        pltpu.async_copy(uemb.at[pl.ds(uidx16[lane], 1), :],
                         ue_t.at[pl.ds(j, 1), :], sem)
        pltpu.async_copy(iemb.at[pl.ds(iidx16[lane], 1), :],
                         ie_t.at[pl.ds(j, 1), :], sem)
      return carry

    lax.fori_loop(0, _CH // 16, issue, 0)

    pltpu.make_async_copy(uemb.at[pl.ds(0, _CH), :], ue_t, sem).wait()
    pltpu.make_async_copy(iemb.at[pl.ds(0, _CH), :], ie_t, sem).wait()

    def chunk(g, carry):
      ub16 = ub_v[pl.ds(off + g * 16, 16)]
      ib16 = ib_v[pl.ds(off + g * 16, 16)]
      for lane in range(16):
        j = g * 16 + lane
        out_t[j, :] = ((ue_t[j, :] + ub16[lane]) * (ie_t[j, :] + ib16[lane])
                       + bvec)
      return carry

    lax.fori_loop(0, _CH // 16, chunk, 0)

    pltpu.sync_copy(out_t, out.at[pl.ds(base + off, _CH)])


@functools.partial(
    pl.kernel,
    out_type=jax.ShapeDtypeStruct((B, K), jnp.float32),
    mesh=plsc.VectorSubcoreMesh(core_axis_name="c", subcore_axis_name="s"),
    scratch_types=[
        pltpu.VMEM((_BPW,), jnp.int32),
        pltpu.VMEM((_BPW,), jnp.int32),
        pltpu.VMEM((_CH, K), jnp.float32),
        pltpu.VMEM((_CH, K), jnp.float32),
        pltpu.VMEM((_BPW,), jnp.float32),
        pltpu.VMEM((_BPW,), jnp.float32),
        pltpu.VMEM((128,), jnp.float32),
        pltpu.VMEM((_CH, K), jnp.float32),
        pltpu.SemaphoreType.DMA,
    ],
)
def _funk(*args):
  _funk_body(*args)


def kernel(user_id, item_id, user_emb, user_bias, item_emb, item_bias, bias):
  bias128 = jnp.broadcast_to(bias.astype(jnp.float32), (128,))
  return _funk(user_id.astype(jnp.int32), item_id.astype(jnp.int32),
               user_emb, user_bias, item_emb, item_bias, bias128)


# speedup vs baseline: 5.9195x; 3.9900x over previous
import functools

import jax
import jax.numpy as jnp
from jax import lax
from jax.experimental import pallas as pl
from jax.experimental.pallas import tpu as pltpu
from jax.experimental.pallas import tpu_sc as plsc

M = 1000000
N = 1000000
K = 16
B = 16384

_NC = 2
_NS = 16
_NW = _NC * _NS
_BPW = B // _NW
_CH = 16


def _funk_body(uid, iid, uembT, ubias, iembT, ibias, bias128, outT,
               uidx_v, iidx_v, wu_v, wi_v, ub_v, ib_v, bias_v, out_t, sem):
  wid = lax.axis_index("s") * _NC + lax.axis_index("c")
  base = wid * _BPW

  pltpu.sync_copy(uid.at[pl.ds(base, _BPW)], uidx_v)
  pltpu.sync_copy(iid.at[pl.ds(base, _BPW)], iidx_v)
  pltpu.sync_copy(bias128, bias_v)

  bcp1 = pltpu.async_copy(ubias.at[uidx_v], ub_v, sem)
  bcp2 = pltpu.async_copy(ibias.at[iidx_v], ib_v, sem)
  bcp1.wait()
  bcp2.wait()

  bvec = bias_v[pl.ds(0, 16)]
  lanes = lax.iota(jnp.int32, 16)

  def chunk(c, carry):
    s0 = c * _CH
    uidx16 = uidx_v[pl.ds(s0, 16)]
    iidx16 = iidx_v[pl.ds(s0, 16)]
    jw_u = lax.shift_right_logical(uidx16, 7)
    jw_i = lax.shift_right_logical(iidx16, 7)
    r_u = lax.bitwise_and(uidx16, 127)
    r_i = lax.bitwise_and(iidx16, 127)

    cps = []
    for lane in range(_CH):
      cu = jw_u[lane] * 128
      ci = jw_i[lane] * 128
      cps.append(pltpu.async_copy(
          uembT.at[pl.ds(0, 8), pl.ds(cu, 128)], wu_v.at[lane, 0], sem))
      cps.append(pltpu.async_copy(
          uembT.at[pl.ds(8, 8), pl.ds(cu, 128)], wu_v.at[lane, 1], sem))
      cps.append(pltpu.async_copy(
          iembT.at[pl.ds(0, 8), pl.ds(ci, 128)], wi_v.at[lane, 0], sem))
      cps.append(pltpu.async_copy(
          iembT.at[pl.ds(8, 8), pl.ds(ci, 128)], wi_v.at[lane, 1], sem))
    for cp in cps:
      cp.wait()

    ub16 = ub_v[pl.ds(s0, 16)]
    ib16 = ib_v[pl.ds(s0, 16)]
    for k in range(K):
      h = jnp.full((16,), k // 8, jnp.int32)
      s = jnp.full((16,), k % 8, jnp.int32)
      ue_k = plsc.load_gather(wu_v, [lanes, h, s, r_u])
      ie_k = plsc.load_gather(wi_v, [lanes, h, s, r_i])
      out_t[k, pl.ds(s0, 16)] = (ue_k + ub16) * (ie_k + ib16) + bvec
    return carry

  lax.fori_loop(0, _BPW // _CH, chunk, 0)

  pltpu.sync_copy(out_t, outT.at[:, pl.ds(base, _BPW)])


@functools.partial(
    pl.kernel,
    out_type=jax.ShapeDtypeStruct((K, B), jnp.float32),
    mesh=plsc.VectorSubcoreMesh(core_axis_name="c", subcore_axis_name="s"),
    scratch_types=[
        pltpu.VMEM((_BPW,), jnp.int32),
        pltpu.VMEM((_BPW,), jnp.int32),
        pltpu.VMEM((_CH, 2, 8, 128), jnp.float32),
        pltpu.VMEM((_CH, 2, 8, 128), jnp.float32),
        pltpu.VMEM((_BPW,), jnp.float32),
        pltpu.VMEM((_BPW,), jnp.float32),
        pltpu.VMEM((128,), jnp.float32),
        pltpu.VMEM((K, _BPW), jnp.float32),
        pltpu.SemaphoreType.DMA,
    ],
    compiler_params=pltpu.CompilerParams(needs_layout_passes=False),
)
def _funk(*args):
  _funk_body(*args)


def kernel(user_id, item_id, user_emb, user_bias, item_emb, item_bias, bias):
  bias128 = jnp.broadcast_to(bias.astype(jnp.float32), (128,))
  outT = _funk(user_id.astype(jnp.int32), item_id.astype(jnp.int32),
               user_emb.T, user_bias, item_emb.T, item_bias, bias128)
  return outT.T
